# baseline (speedup 1.0000x reference)
import jax
import jax.numpy as jnp
from jax.experimental import pallas as pl
from jax.experimental.pallas import tpu as pltpu

IN_DIM = 768
E_DIM = 32
N_EMB = 256
N_LEVELS = 3
MU = 0.25
ALPHA = 0.1
QUANT_W = 1.0
TEMP = 0.1
BATCH = 4096

TB = 1024
NT = BATCH // TB


def _mm_t(a, w):
    return jax.lax.dot_general(a, w, (((1,), (1,)), ((), ())),
                               preferred_element_type=jnp.float32)


def _body(x_ref, cf_ref,
          ew0, ew1, ew2, ew3, ew4,
          eb0, eb1, eb2, eb3, eb4,
          dw0, dw1, dw2, dw3, dw4,
          db0, db1, db2, db3, db4,
          cb_ref,
          xr_ref, z_ref, zq_ref, idx_ref, loss_ref,
          qn_ref, cn_ref, c2_ref, rs_ref, qs_ref, cf_acc_ref):
    i = pl.program_id(0)

    @pl.when(i == 0)
    def _norms():
        cb = cb_ref[...]
        c2_ref[0:N_LEVELS, :] = jnp.sum(cb * cb, axis=2)

    @pl.when(i < NT)
    def _fwd():
        rows = pl.ds(i * TB, TB)
        x = x_ref[...]

        h = x
        enc = ((ew0, eb0), (ew1, eb1), (ew2, eb2), (ew3, eb3), (ew4, eb4))
        for k, (w, b) in enumerate(enc):
            h = _mm_t(h, w[...]) + b[...]
            if k != len(enc) - 1:
                h = jnp.maximum(h, 0.0)
        z = h
        z_ref[...] = z

        cb = cb_ref[...]
        col = jax.lax.broadcasted_iota(jnp.int32, (TB, N_EMB), 1)
        residual = z
        zq = jnp.zeros_like(z)
        qsse = jnp.float32(0.0)
        idx_cols = []
        for l in range(N_LEVELS):
            cbl = cb[l]
            r2 = jnp.sum(residual * residual, axis=1, keepdims=True)
            if l > 0:
                qsse = qsse + jnp.sum(r2)
            c2 = c2_ref[l:l + 1, :]
            d = r2 + _mm_t(residual * (-2.0), cbl) + c2
            dmin = jnp.min(d, axis=1, keepdims=True)
            idx = jnp.min(jnp.where(d == dmin, col, N_EMB), axis=1)
            onehot = (col == idx[:, None]).astype(jnp.float32)
            e = jax.lax.dot_general(onehot, cbl, (((1,), (0,)), ((), ())),
                                    precision=jax.lax.Precision.HIGHEST,
                                    preferred_element_type=jnp.float32)
            zq = zq + e
            residual = residual - e
            idx_cols.append(idx[:, None].astype(jnp.int32))
        idx_ref[...] = jnp.concatenate(idx_cols, axis=1)
        qsse = qsse + jnp.sum(residual * residual)
        zq_ref[...] = zq

        h = zq
        dec = ((dw0, db0), (dw1, db1), (dw2, db2), (dw3, db3), (dw4, db4))
        for k, (w, b) in enumerate(dec):
            wv = w[...]
            if wv.dtype == jnp.bfloat16:
                h = jax.lax.dot_general(h.astype(jnp.bfloat16), wv,
                                        (((1,), (1,)), ((), ())),
                                        preferred_element_type=jnp.float32)
                h = h + b[...]
            else:
                h = _mm_t(h, wv) + b[...]
            if k != len(dec) - 1:
                h = jnp.maximum(h, 0.0)
        xr = h
        xr_ref[...] = xr
        rsse = jnp.sum((xr - x) ** 2)

        qn_ref[rows, :] = zq / (jnp.sqrt(jnp.sum(zq * zq, axis=1,
                                                 keepdims=True)) + 1e-12)
        cf = cf_ref[...]
        cn_ref[rows, :] = cf / (jnp.sqrt(jnp.sum(cf * cf, axis=1,
                                                 keepdims=True)) + 1e-12)

        @pl.when(i == 0)
        def _():
            rs_ref[0, 0] = rsse
            qs_ref[0, 0] = qsse

        @pl.when(i != 0)
        def _():
            rs_ref[0, 0] += rsse
            qs_ref[0, 0] += qsse

    @pl.when(i >= NT)
    def _cf():
        j = i - NT
        rows = pl.ds(j * TB, TB)
        qn = qn_ref[rows, :]
        cn = cn_ref[...]
        sim = jnp.dot((qn * (1.0 / TEMP)).astype(jnp.bfloat16),
                      cn.astype(jnp.bfloat16).T,
                      preferred_element_type=jnp.float32)
        lse = jnp.log(jnp.sum(jnp.exp(sim), axis=1))
        pos = jnp.sum(qn * cn_ref[rows, :], axis=1) * (1.0 / TEMP)
        s = jnp.sum(lse - pos)

        @pl.when(j == 0)
        def _():
            cf_acc_ref[0, 0] = s

        @pl.when(j != 0)
        def _():
            cf_acc_ref[0, 0] += s

    @pl.when(i == 2 * NT - 1)
    def _finalize():
        recon = rs_ref[0, 0] / (BATCH * IN_DIM)
        quant = (1.0 + MU) * qs_ref[0, 0] / (BATCH * E_DIM)
        cfl = cf_acc_ref[0, 0] / BATCH
        total = recon + QUANT_W * quant + ALPHA * cfl
        lane = jax.lax.broadcasted_iota(jnp.int32, (1, 128), 1)
        v = jnp.where(lane == 0, recon, 0.0)
        v = jnp.where(lane == 1, quant, v)
        v = jnp.where(lane == 2, cfl, v)
        v = jnp.where(lane == 3, total, v)
        loss_ref[...] = v


def kernel(x, cf_emb, enc_Ws, enc_bs, dec_Ws, dec_bs, codebooks):
    ebr = [b.reshape(1, -1) for b in enc_bs]
    dbr = [b.reshape(1, -1) for b in dec_bs]
    dec_Ws = list(dec_Ws[:3]) + [w.astype(jnp.bfloat16) for w in dec_Ws[3:]]

    def full(a):
        return pl.BlockSpec(a.shape, lambda i: (0,) * a.ndim)

    def tiled(width):
        return pl.BlockSpec((TB, width), lambda i: (jnp.minimum(i, NT - 1), 0))

    in_specs = ([tiled(IN_DIM), tiled(E_DIM)]
                + [full(w) for w in enc_Ws] + [full(b) for b in ebr]
                + [full(w) for w in dec_Ws] + [full(b) for b in dbr]
                + [full(codebooks)])
    out_shape = [
        jax.ShapeDtypeStruct((BATCH, IN_DIM), jnp.float32),
        jax.ShapeDtypeStruct((BATCH, E_DIM), jnp.float32),
        jax.ShapeDtypeStruct((BATCH, E_DIM), jnp.float32),
        jax.ShapeDtypeStruct((BATCH, N_LEVELS), jnp.int32),
        jax.ShapeDtypeStruct((1, 128), jnp.float32),
    ]
    out_specs = [
        tiled(IN_DIM), tiled(E_DIM), tiled(E_DIM),
        tiled(N_LEVELS),
        pl.BlockSpec((1, 128), lambda i: (0, 0)),
    ]
    (xr, z, zq, indices, losses) = pl.pallas_call(
        _body, grid=(2 * NT,), in_specs=in_specs,
        out_specs=out_specs, out_shape=out_shape,
        scratch_shapes=[pltpu.VMEM((BATCH, E_DIM), jnp.float32),
                        pltpu.VMEM((BATCH, E_DIM), jnp.float32),
                        pltpu.VMEM((8, N_EMB), jnp.float32),
                        pltpu.SMEM((1, 1), jnp.float32),
                        pltpu.SMEM((1, 1), jnp.float32),
                        pltpu.SMEM((1, 1), jnp.float32)],
    )(x, cf_emb, *enc_Ws, *ebr, *dec_Ws, *dbr, codebooks)

    recon_loss = losses[0, 0]
    quant_loss = losses[0, 1]
    cf_loss = losses[0, 2]
    total_loss = losses[0, 3]
    div_loss = jnp.float32(0.0)
    return (xr, z, zq, indices, recon_loss, quant_loss, div_loss,
            cf_loss, total_loss)


# speedup vs baseline: 1.0379x; 1.0379x over previous
import jax
import jax.numpy as jnp
from jax.experimental import pallas as pl
from jax.experimental.pallas import tpu as pltpu

IN_DIM = 768
E_DIM = 32
N_EMB = 256
N_LEVELS = 3
MU = 0.25
ALPHA = 0.1
QUANT_W = 1.0
TEMP = 0.1
BATCH = 4096

TB = 1024
NT = BATCH // TB


def _mm_t(a, w):
    return jax.lax.dot_general(a, w, (((1,), (1,)), ((), ())),
                               preferred_element_type=jnp.float32)


def _body(x_ref, cf_ref,
          ew0, ew1, ew2, ew3, ew4,
          eb0, eb1, eb2, eb3, eb4,
          dw0, dw1, dw2, dw3, dw4,
          db0, db1, db2, db3, db4,
          cb_ref,
          xr_ref, z_ref, zq_ref, idx_ref, loss_ref,
          qn_ref, cn_ref, c2_ref, rs_ref, qs_ref, cf_acc_ref):
    i = pl.program_id(0)

    @pl.when(i == 0)
    def _norms():
        cb = cb_ref[...]
        c2_ref[0:N_LEVELS, :] = jnp.sum(cb * cb, axis=2)

    @pl.when(i < NT)
    def _fwd():
        rows = pl.ds(i * TB, TB)
        x = x_ref[...]

        h = x
        enc = ((ew0, eb0), (ew1, eb1), (ew2, eb2), (ew3, eb3), (ew4, eb4))
        for k, (w, b) in enumerate(enc):
            h = _mm_t(h, w[...]) + b[...]
            if k != len(enc) - 1:
                h = jnp.maximum(h, 0.0)
        z = h
        z_ref[...] = z

        cb = cb_ref[...]
        col = jax.lax.broadcasted_iota(jnp.int32, (TB, N_EMB), 1)
        residual = z
        zq = jnp.zeros_like(z)
        qsse = jnp.float32(0.0)
        idx_cols = []
        for l in range(N_LEVELS):
            cbl = cb[l]
            r2 = jnp.sum(residual * residual, axis=1, keepdims=True)
            if l > 0:
                qsse = qsse + jnp.sum(r2)
            c2 = c2_ref[l:l + 1, :]
            d = r2 + _mm_t(residual * (-2.0), cbl) + c2
            dmin = jnp.min(d, axis=1, keepdims=True)
            idx = jnp.min(jnp.where(d == dmin, col, N_EMB), axis=1)
            onehot = (col == idx[:, None]).astype(jnp.float32)
            e = jax.lax.dot_general(onehot, cbl, (((1,), (0,)), ((), ())),
                                    precision=jax.lax.Precision.HIGHEST,
                                    preferred_element_type=jnp.float32)
            zq = zq + e
            residual = residual - e
            idx_cols.append(idx[:, None].astype(jnp.int32))
        idx_ref[...] = jnp.concatenate(idx_cols, axis=1)
        qsse = qsse + jnp.sum(residual * residual)
        zq_ref[...] = zq

        h = zq
        dec = ((dw0, db0), (dw1, db1), (dw2, db2), (dw3, db3), (dw4, db4))
        for k, (w, b) in enumerate(dec):
            h = _mm_t(h, w[...]) + b[...]
            if k != len(dec) - 1:
                h = jnp.maximum(h, 0.0)
        xr = h
        xr_ref[...] = xr
        rsse = jnp.sum((xr - x) ** 2)

        qn_ref[rows, :] = zq / (jnp.sqrt(jnp.sum(zq * zq, axis=1,
                                                 keepdims=True)) + 1e-12)
        cf = cf_ref[...]
        cn_ref[rows, :] = cf / (jnp.sqrt(jnp.sum(cf * cf, axis=1,
                                                 keepdims=True)) + 1e-12)

        @pl.when(i == 0)
        def _():
            rs_ref[0, 0] = rsse
            qs_ref[0, 0] = qsse

        @pl.when(i != 0)
        def _():
            rs_ref[0, 0] += rsse
            qs_ref[0, 0] += qsse

    @pl.when(i >= NT)
    def _cf():
        j = i - NT
        rows = pl.ds(j * TB, TB)
        qn = qn_ref[rows, :]
        cn = cn_ref[...]
        sim = jnp.dot((qn * (1.0 / TEMP)).astype(jnp.bfloat16),
                      cn.astype(jnp.bfloat16).T,
                      preferred_element_type=jnp.float32)
        lse = jnp.log(jnp.sum(jnp.exp(sim), axis=1))
        pos = jnp.sum(qn * cn_ref[rows, :], axis=1) * (1.0 / TEMP)
        s = jnp.sum(lse - pos)

        @pl.when(j == 0)
        def _():
            cf_acc_ref[0, 0] = s

        @pl.when(j != 0)
        def _():
            cf_acc_ref[0, 0] += s

    @pl.when(i == 2 * NT - 1)
    def _finalize():
        recon = rs_ref[0, 0] / (BATCH * IN_DIM)
        quant = (1.0 + MU) * qs_ref[0, 0] / (BATCH * E_DIM)
        cfl = cf_acc_ref[0, 0] / BATCH
        total = recon + QUANT_W * quant + ALPHA * cfl
        lane = jax.lax.broadcasted_iota(jnp.int32, (1, 128), 1)
        v = jnp.where(lane == 0, recon, 0.0)
        v = jnp.where(lane == 1, quant, v)
        v = jnp.where(lane == 2, cfl, v)
        v = jnp.where(lane == 3, total, v)
        loss_ref[...] = v


def kernel(x, cf_emb, enc_Ws, enc_bs, dec_Ws, dec_bs, codebooks):
    ebr = [b.reshape(1, -1) for b in enc_bs]
    dbr = [b.reshape(1, -1) for b in dec_bs]

    def full(a):
        return pl.BlockSpec(a.shape, lambda i: (0,) * a.ndim)

    def tiled(width):
        return pl.BlockSpec((TB, width), lambda i: (jnp.minimum(i, NT - 1), 0))

    in_specs = ([tiled(IN_DIM), tiled(E_DIM)]
                + [full(w) for w in enc_Ws] + [full(b) for b in ebr]
                + [full(w) for w in dec_Ws] + [full(b) for b in dbr]
                + [full(codebooks)])
    out_shape = [
        jax.ShapeDtypeStruct((BATCH, IN_DIM), jnp.float32),
        jax.ShapeDtypeStruct((BATCH, E_DIM), jnp.float32),
        jax.ShapeDtypeStruct((BATCH, E_DIM), jnp.float32),
        jax.ShapeDtypeStruct((BATCH, N_LEVELS), jnp.int32),
        jax.ShapeDtypeStruct((1, 128), jnp.float32),
    ]
    out_specs = [
        tiled(IN_DIM), tiled(E_DIM), tiled(E_DIM),
        tiled(N_LEVELS),
        pl.BlockSpec((1, 128), lambda i: (0, 0)),
    ]
    (xr, z, zq, indices, losses) = pl.pallas_call(
        _body, grid=(2 * NT,), in_specs=in_specs,
        out_specs=out_specs, out_shape=out_shape,
        scratch_shapes=[pltpu.VMEM((BATCH, E_DIM), jnp.float32),
                        pltpu.VMEM((BATCH, E_DIM), jnp.float32),
                        pltpu.VMEM((8, N_EMB), jnp.float32),
                        pltpu.SMEM((1, 1), jnp.float32),
                        pltpu.SMEM((1, 1), jnp.float32),
                        pltpu.SMEM((1, 1), jnp.float32)],
    )(x, cf_emb, *enc_Ws, *ebr, *dec_Ws, *dbr, codebooks)

    recon_loss = losses[0, 0]
    quant_loss = losses[0, 1]
    cf_loss = losses[0, 2]
    total_loss = losses[0, 3]
    div_loss = jnp.float32(0.0)
    return (xr, z, zq, indices, recon_loss, quant_loss, div_loss,
            cf_loss, total_loss)


# speedup vs baseline: 1.0606x; 1.0219x over previous
import jax
import jax.numpy as jnp
from jax.experimental import pallas as pl
from jax.experimental.pallas import tpu as pltpu

IN_DIM = 768
E_DIM = 32
N_EMB = 256
N_LEVELS = 3
MU = 0.25
ALPHA = 0.1
QUANT_W = 1.0
TEMP = 0.1
BATCH = 4096

TB = 1024
NT = BATCH // TB


def _mm_t(a, w):
    return jax.lax.dot_general(a, w, (((1,), (1,)), ((), ())),
                               preferred_element_type=jnp.float32)


def _body(x_ref, cf_ref,
          ew0, ew1, ew2, ew3, ew4,
          eb0, eb1, eb2, eb3, eb4,
          dw0, dw1, dw2, dw3, dw4,
          db0, db1, db2, db3, db4,
          cb_ref,
          xr_ref, z_ref, zq_ref, idx_ref, loss_ref,
          cn_ref, cnb_ref, c2_ref, rs_ref, qs_ref, cf_acc_ref):
    i = pl.program_id(0)

    @pl.when(i == 0)
    def _prep():
        cb = cb_ref[...]
        c2_ref[0:N_LEVELS, :] = jnp.sum(cb * cb, axis=2)
        cf = cf_ref[...]
        cn = cf / (jnp.sqrt(jnp.sum(cf * cf, axis=1, keepdims=True)) + 1e-12)
        cn_ref[...] = cn
        cnb_ref[...] = cn.astype(jnp.bfloat16)

    rows = pl.ds(i * TB, TB)
    x = x_ref[...]

    h = x
    enc = ((ew0, eb0), (ew1, eb1), (ew2, eb2), (ew3, eb3), (ew4, eb4))
    for k, (w, b) in enumerate(enc):
        h = _mm_t(h, w[...]) + b[...]
        if k != len(enc) - 1:
            h = jnp.maximum(h, 0.0)
    z = h
    z_ref[...] = z

    cb = cb_ref[...]
    col = jax.lax.broadcasted_iota(jnp.int32, (TB, N_EMB), 1)
    residual = z
    zq = jnp.zeros_like(z)
    qsse = jnp.float32(0.0)
    idx_cols = []
    for l in range(N_LEVELS):
        cbl = cb[l]
        r2 = jnp.sum(residual * residual, axis=1, keepdims=True)
        if l > 0:
            qsse = qsse + jnp.sum(r2)
        c2 = c2_ref[l:l + 1, :]
        d = r2 + _mm_t(residual * (-2.0), cbl) + c2
        dmin = jnp.min(d, axis=1, keepdims=True)
        idx = jnp.min(jnp.where(d == dmin, col, N_EMB), axis=1)
        onehot = (col == idx[:, None]).astype(jnp.float32)
        e = jax.lax.dot_general(onehot, cbl, (((1,), (0,)), ((), ())),
                                precision=jax.lax.Precision.HIGHEST,
                                preferred_element_type=jnp.float32)
        zq = zq + e
        residual = residual - e
        idx_cols.append(idx[:, None].astype(jnp.int32))
    idx_ref[...] = jnp.concatenate(idx_cols, axis=1)
    qsse = qsse + jnp.sum(residual * residual)
    zq_ref[...] = zq

    h = zq
    dec = ((dw0, db0), (dw1, db1), (dw2, db2), (dw3, db3), (dw4, db4))
    for k, (w, b) in enumerate(dec):
        h = _mm_t(h, w[...]) + b[...]
        if k != len(dec) - 1:
            h = jnp.maximum(h, 0.0)
    xr = h
    xr_ref[...] = xr
    rsse = jnp.sum((xr - x) ** 2)

    qn = zq / (jnp.sqrt(jnp.sum(zq * zq, axis=1, keepdims=True)) + 1e-12)
    sim = jax.lax.dot_general((qn * (1.0 / TEMP)).astype(jnp.bfloat16),
                              cnb_ref[...], (((1,), (1,)), ((), ())),
                              preferred_element_type=jnp.float32)
    lse = jnp.log(jnp.sum(jnp.exp(sim), axis=1))
    pos = jnp.sum(qn * cn_ref[rows, :], axis=1) * (1.0 / TEMP)
    cfs = jnp.sum(lse - pos)

    @pl.when(i == 0)
    def _():
        rs_ref[0, 0] = rsse
        qs_ref[0, 0] = qsse
        cf_acc_ref[0, 0] = cfs

    @pl.when(i != 0)
    def _():
        rs_ref[0, 0] += rsse
        qs_ref[0, 0] += qsse
        cf_acc_ref[0, 0] += cfs

    @pl.when(i == NT - 1)
    def _finalize():
        recon = rs_ref[0, 0] / (BATCH * IN_DIM)
        quant = (1.0 + MU) * qs_ref[0, 0] / (BATCH * E_DIM)
        cfl = cf_acc_ref[0, 0] / BATCH
        total = recon + QUANT_W * quant + ALPHA * cfl
        lane = jax.lax.broadcasted_iota(jnp.int32, (1, 128), 1)
        v = jnp.where(lane == 0, recon, 0.0)
        v = jnp.where(lane == 1, quant, v)
        v = jnp.where(lane == 2, cfl, v)
        v = jnp.where(lane == 3, total, v)
        loss_ref[...] = v


def kernel(x, cf_emb, enc_Ws, enc_bs, dec_Ws, dec_bs, codebooks):
    ebr = [b.reshape(1, -1) for b in enc_bs]
    dbr = [b.reshape(1, -1) for b in dec_bs]

    def full(a):
        return pl.BlockSpec(a.shape, lambda i: (0,) * a.ndim)

    def tiled(width):
        return pl.BlockSpec((TB, width), lambda i: (i, 0))

    in_specs = ([tiled(IN_DIM), full(cf_emb)]
                + [full(w) for w in enc_Ws] + [full(b) for b in ebr]
                + [full(w) for w in dec_Ws] + [full(b) for b in dbr]
                + [full(codebooks)])
    out_shape = [
        jax.ShapeDtypeStruct((BATCH, IN_DIM), jnp.float32),
        jax.ShapeDtypeStruct((BATCH, E_DIM), jnp.float32),
        jax.ShapeDtypeStruct((BATCH, E_DIM), jnp.float32),
        jax.ShapeDtypeStruct((BATCH, N_LEVELS), jnp.int32),
        jax.ShapeDtypeStruct((1, 128), jnp.float32),
    ]
    out_specs = [
        tiled(IN_DIM), tiled(E_DIM), tiled(E_DIM),
        tiled(N_LEVELS),
        pl.BlockSpec((1, 128), lambda i: (0, 0)),
    ]
    (xr, z, zq, indices, losses) = pl.pallas_call(
        _body, grid=(NT,), in_specs=in_specs,
        out_specs=out_specs, out_shape=out_shape,
        scratch_shapes=[pltpu.VMEM((BATCH, E_DIM), jnp.float32),
                        pltpu.VMEM((BATCH, E_DIM), jnp.bfloat16),
                        pltpu.VMEM((8, N_EMB), jnp.float32),
                        pltpu.SMEM((1, 1), jnp.float32),
                        pltpu.SMEM((1, 1), jnp.float32),
                        pltpu.SMEM((1, 1), jnp.float32)],
    )(x, cf_emb, *enc_Ws, *ebr, *dec_Ws, *dbr, codebooks)

    recon_loss = losses[0, 0]
    quant_loss = losses[0, 1]
    cf_loss = losses[0, 2]
    total_loss = losses[0, 3]
    div_loss = jnp.float32(0.0)
    return (xr, z, zq, indices, recon_loss, quant_loss, div_loss,
            cf_loss, total_loss)
